# Initial kernel scaffold; baseline (speedup 1.0000x reference)
import jax
import jax.numpy as jnp
from jax.experimental import pallas as pl


def kernel(layer, step, evidence, event_count, bank_evidence, bank_step, bank_event_count):
    raise NotImplementedError("write your pallas kernel here")


# speedup vs baseline: 1.1244x; 1.1244x over previous
import jax
import jax.numpy as jnp
from jax.experimental import pallas as pl
from jax.experimental.pallas import tpu as pltpu

L, T, D = 32, 1024, 1024


def _update_kernel(layer_ref, step_ref, ec_ref, ev_ref, bev_in_ref, bstep_ref,
                   bec_ref, bev_out_ref, bstep_out_ref, bec_out_ref, sem):
    del bev_in_ref
    layer = layer_ref[0]
    step = step_ref[0]
    ec = ec_ref[0]

    slots = bstep_ref[pl.ds(layer, 1), :]
    col = jax.lax.broadcasted_iota(jnp.int32, (1, T), 1)
    is_empty = slots == -1
    has_empty = jnp.any(is_empty)
    first_empty = jnp.min(jnp.where(is_empty, col, T))
    min_val = jnp.min(slots)
    oldest = jnp.min(jnp.where(slots == min_val, col, T))
    slot = jnp.where(has_empty, first_empty, oldest)

    row_iota = jax.lax.broadcasted_iota(jnp.int32, (L, T), 0)
    col_iota = jax.lax.broadcasted_iota(jnp.int32, (L, T), 1)
    hit = (row_iota == layer) & (col_iota == slot)
    bstep_out_ref[...] = jnp.where(hit, step, bstep_ref[...])
    bec_out_ref[...] = jnp.where(hit, ec, bec_ref[...])

    copy = pltpu.make_async_copy(ev_ref.at[0], bev_out_ref.at[layer, slot], sem)
    copy.start()
    copy.wait()


def kernel(layer, step, evidence, event_count, bank_evidence, bank_step,
           bank_event_count):
    layer_s = jnp.asarray(layer, jnp.int32).reshape(1)
    step_s = jnp.asarray(step, bank_step.dtype).reshape(1)
    ec_s = jnp.asarray(event_count, bank_event_count.dtype).reshape(1)
    ev2 = evidence.astype(bank_evidence.dtype).reshape(1, D)

    return pl.pallas_call(
        _update_kernel,
        out_shape=(
            jax.ShapeDtypeStruct(bank_evidence.shape, bank_evidence.dtype),
            jax.ShapeDtypeStruct(bank_step.shape, bank_step.dtype),
            jax.ShapeDtypeStruct(bank_event_count.shape, bank_event_count.dtype),
        ),
        in_specs=[
            pl.BlockSpec(memory_space=pltpu.MemorySpace.SMEM),
            pl.BlockSpec(memory_space=pltpu.MemorySpace.SMEM),
            pl.BlockSpec(memory_space=pltpu.MemorySpace.SMEM),
            pl.BlockSpec(memory_space=pltpu.MemorySpace.VMEM),
            pl.BlockSpec(memory_space=pltpu.MemorySpace.HBM),
            pl.BlockSpec(memory_space=pltpu.MemorySpace.VMEM),
            pl.BlockSpec(memory_space=pltpu.MemorySpace.VMEM),
        ],
        out_specs=(
            pl.BlockSpec(memory_space=pltpu.MemorySpace.HBM),
            pl.BlockSpec(memory_space=pltpu.MemorySpace.VMEM),
            pl.BlockSpec(memory_space=pltpu.MemorySpace.VMEM),
        ),
        input_output_aliases={4: 0},
        scratch_shapes=[pltpu.SemaphoreType.DMA],
    )(layer_s, step_s, ec_s, ev2, bank_evidence, bank_step, bank_event_count)
